# Initial kernel scaffold; baseline (speedup 1.0000x reference)
import jax
import jax.numpy as jnp
from jax.experimental import pallas as pl


def kernel(x, y):
    raise NotImplementedError("write your pallas kernel here")


# speedup vs baseline: 1.6378x; 1.6378x over previous
import jax
import jax.numpy as jnp
from jax.experimental import pallas as pl


_CHUNK = 512


def _chamfer_kernel(xp_ref, y_ref, out_x_ref, out_y_ref):
    n = y_ref.shape[1]
    d_dims = y_ref.shape[0]
    n_chunks = n // _CHUNK

    ymin = jnp.full((n,), jnp.inf, dtype=jnp.float32)
    for i in range(n_chunks):
        acc = jnp.zeros((_CHUNK, n), dtype=jnp.float32)
        for d in range(d_dims):
            xc = xp_ref[pl.ds(i * _CHUNK, _CHUNK), d : d + 1]
            yr = y_ref[d : d + 1, :]
            diff = xc - yr
            acc = acc + diff * diff
        out_x_ref[0, pl.ds(i * _CHUNK, _CHUNK)] = jnp.min(acc, axis=1)
        ymin = jnp.minimum(ymin, jnp.min(acc, axis=0))
    out_y_ref[0, :] = ymin


def kernel(x, y):
    b, d, n = x.shape
    xp = jnp.transpose(x, (0, 2, 1))

    out_x, out_y = pl.pallas_call(
        _chamfer_kernel,
        grid=(b,),
        in_specs=[
            pl.BlockSpec((None, n, d), lambda i: (i, 0, 0)),
            pl.BlockSpec((None, d, n), lambda i: (i, 0, 0)),
        ],
        out_specs=[
            pl.BlockSpec((None, 1, n), lambda i: (i, 0, 0)),
            pl.BlockSpec((None, 1, n), lambda i: (i, 0, 0)),
        ],
        out_shape=[
            jax.ShapeDtypeStruct((b, 1, n), jnp.float32),
            jax.ShapeDtypeStruct((b, 1, n), jnp.float32),
        ],
    )(xp, y)

    return jnp.mean(out_x) + jnp.mean(out_y)
